# baseline (speedup 1.0000x reference)
import functools

import numpy as np
import jax
import jax.numpy as jnp
from jax import lax
from jax.experimental import pallas as pl
from jax.experimental.pallas import tpu as pltpu, tpu_sc as plsc

H = 384
W = 384
B = 4
HW = H * W
G = int(HW * 0.15)
LANES = 128
GROW = 176
HGROW = GROW // 2
NW = 32
NROWS = 288
ROWS = NROWS // NW
TW = 2 * B
NSUB = 16
PIXSUB = HW // NSUB
PP = PIXSUB // 4

_DELTA_Z = 1e-4
_DELTA_COS = 0.867
_DXYZ = 0.005


def _static_consts():
    rng = np.random.RandomState(0)
    ps = []
    for _ in range(3):
        p = rng.choice(HW, G, replace=True)
        rng.shuffle(p)
        ps.append(p.astype(np.int64))
    pixk = np.zeros((3, GROW, LANES), np.int32)
    cu = np.zeros((3, GROW * LANES), np.float32)
    cv = np.zeros((3, GROW * LANES), np.float32)
    pm = np.zeros((GROW * LANES,), np.float32)
    pm[:G] = 1.0
    for k in range(3):
        p = ps[k]
        pk = np.zeros((GROW * LANES,), np.int32)
        pk[:G] = p.astype(np.int32)
        pixk[k] = pk.reshape(GROW, LANES)
        cu[k, :G] = (p % W).astype(np.float32) - float(W // 2)
        cv[k, :G] = (p // W).astype(np.float32) - float(H // 2)
    cu = cu.reshape(3, GROW, LANES)
    cv = cv.reshape(3, GROW, LANES)
    pm = pm.reshape(GROW, LANES)
    halves = []
    for hh in range(2):
        sl = slice(hh * HGROW, (hh + 1) * HGROW)
        pix = np.zeros((NROWS, LANES), np.int32)
        pix[:3 * HGROW] = pixk[:, sl].reshape(3 * HGROW, LANES)
        halves.append((pix, np.ascontiguousarray(cu[:, sl]),
                       np.ascontiguousarray(cv[:, sl]),
                       np.ascontiguousarray(pm[sl])))
    return halves


(_PIX1, _CU1, _CV1, _PM1), (_PIX2, _CU2, _CV2, _PM2) = _static_consts()


@functools.lru_cache(maxsize=2)
def _get_sc_gather(build_table):
    mesh = plsc.VectorSubcoreMesh(core_axis_name="c", subcore_axis_name="s",
                                  num_cores=2, num_subcores=16)
    out_type = jax.ShapeDtypeStruct((TW, NROWS, LANES), jnp.float32)
    if build_table:
        out_type = (out_type, jax.ShapeDtypeStruct((2, HW, TW), jnp.float32))

    @functools.partial(
        pl.kernel,
        out_type=out_type,
        mesh=mesh,
        scratch_types=[
            pltpu.VMEM((2, TW, PP), jnp.float32),
            pltpu.VMEM((PP, TW), jnp.float32),
            pltpu.VMEM((ROWS, LANES), jnp.int32),
            pltpu.VMEM((ROWS, LANES, TW), jnp.float32),
            pltpu.VMEM((TW, ROWS, LANES), jnp.float32),
            pltpu.SemaphoreType.DMA,
            pltpu.SemaphoreType.DMA,
        ],
        compiler_params=pltpu.CompilerParams(use_tc_tiling_on_sc=False,
                                             needs_layout_passes=False),
    )
    def _sc_gather(*refs):
        if build_table:
            (gt_hbm, pr_hbm, idx_hbm, out_hbm, tab_hbm,
             chunks, tblh, idx_v, buf, outv, sem, sem2) = refs
        else:
            (tab_hbm, idx_hbm, out_hbm,
             chunks, tblh, idx_v, buf, outv, sem, sem2) = refs
        cid = lax.axis_index("c")
        sid = lax.axis_index("s")
        wid = sid * 2 + cid
        rowbase = wid * ROWS
        idx_h = pltpu.async_copy(idx_hbm.at[pl.ds(rowbase, ROWS)], idx_v,
                                 sem2)
        lane = lax.iota(jnp.int32, 16)
        if build_table:
            tconst = [jnp.full((16,), t, jnp.int32) for t in range(TW)]

            def fire_loads(p, sel):
                base = sid * PIXSUB + p * PP
                hs = []
                for t in range(TW):
                    src = gt_hbm if t < B else pr_hbm
                    hs.append(pltpu.async_copy(
                        src.at[t % B, pl.ds(base, PP)], chunks.at[sel, t],
                        sem))
                return hs

            pend = fire_loads(0, 0)
            for p in range(4):
                cur = p % 2
                for hh in pend:
                    hh.wait()
                if p < 3:
                    pend = fire_loads(p + 1, 1 - cur)

                def interleave(i, carry):
                    pvec = lane + 16 * i
                    for t in range(TW):
                        vals = chunks[cur, t, pl.ds(16 * i, 16)]
                        plsc.store_scatter(tblh, [pvec, tconst[t]], vals)
                    return carry

                lax.fori_loop(0, PP // 16, interleave, jnp.int32(0))
                pltpu.sync_copy(
                    tblh, tab_hbm.at[cid, pl.ds(sid * PIXSUB + p * PP, PP)])
            idx_h.wait()
            plsc.subcore_barrier()
        else:
            idx_h.wait()
        handles = []
        for r in range(ROWS):
            handles.append(
                pltpu.async_copy(tab_hbm.at[cid].at[idx_v.at[r]],
                                 buf.at[r], sem))
        for h in handles:
            h.wait()

        def deint(r, carry):
            rr = jnp.full((16,), r, jnp.int32)
            for t in range(TW):
                tt = jnp.full((16,), t, jnp.int32)
                for j in range(LANES // 16):
                    ll = lane + (16 * j)
                    vals = plsc.load_gather(buf, [rr, ll, tt])
                    outv[t, r, pl.ds(16 * j, 16)] = vals
            return carry

        lax.fori_loop(0, ROWS, deint, jnp.int32(0))
        pltpu.sync_copy(outv, out_hbm.at[:, pl.ds(rowbase, ROWS)])

    return _sc_gather


def _group_vbits(dep, cx, cy, pm, b):
    pmv = pm[...] > 0
    cxs = [cx[k] for k in range(3)]
    cys = [cy[k] for k in range(3)]
    dg = [dep[b, k * HGROW:(k + 1) * HGROW] for k in range(3)]
    dp = [dep[B + b, k * HGROW:(k + 1) * HGROW] for k in range(3)]

    gx = [cxs[k] * jnp.abs(dg[k]) for k in range(3)]
    gy = [cys[k] * jnp.abs(dg[k]) for k in range(3)]
    gz = dg

    pX = [cxs[k] * jnp.abs(dp[k]) for k in range(3)]
    pY = [cys[k] * jnp.abs(dp[k]) for k in range(3)]
    pZ = list(dp)
    cz = [dp[j] == 0.0 for j in range(3)]
    pX = [jnp.where(cz[0], jnp.float32(1e-4), pX[k]) for k in range(3)]
    pY = [jnp.where(cz[1], jnp.float32(1e-4), pY[k]) for k in range(3)]
    pZ = [jnp.where(cz[2], jnp.float32(1e-4), pZ[k]) for k in range(3)]

    pairs = ((0, 1), (0, 2), (1, 2))
    e = [(gx[j] - gx[i], gy[j] - gy[i], gz[j] - gz[i]) for (i, j) in pairs]

    def dot(a, b_):
        return a[0] * b_[0] + a[1] * b_[1] + a[2] * b_[2]

    d00 = dot(e[0], e[0])
    d11 = dot(e[1], e[1])
    d22 = dot(e[2], e[2])
    d01 = dot(e[0], e[1])
    d02 = dot(e[0], e[2])
    d12 = dot(e[1], e[2])
    n0 = jnp.sqrt(d00)
    n1 = jnp.sqrt(d11)
    n2 = jnp.sqrt(d22)
    eps = jnp.float32(1e-8)

    def hit(num, na, nb):
        return (jnp.abs(num / (na * nb + eps)) > _DELTA_COS).astype(jnp.int32)

    cnt = (hit(d00, n0, n0) + hit(d11, n1, n1) + hit(d22, n2, n2)
           + 2 * hit(d01, n0, n1) + 2 * hit(d02, n0, n2)
           + 2 * hit(d12, n1, n2))
    mask_cos = cnt > 3
    mask_pad = (gz[0] > _DELTA_Z) & (gz[1] > _DELTA_Z) & (gz[2] > _DELTA_Z)
    mx = ((jnp.abs(e[0][0]) < _DXYZ) | (jnp.abs(e[1][0]) < _DXYZ)
          | (jnp.abs(e[2][0]) < _DXYZ))
    my = ((jnp.abs(e[0][1]) < _DXYZ) | (jnp.abs(e[1][1]) < _DXYZ)
          | (jnp.abs(e[2][1]) < _DXYZ))
    mz = ((jnp.abs(e[0][2]) < _DXYZ) | (jnp.abs(e[1][2]) < _DXYZ)
          | (jnp.abs(e[2][2]) < _DXYZ))
    mask = mask_pad & jnp.logical_not((mx & my & mz) | mask_cos) & pmv

    def cross(a, b_):
        return (a[1] * b_[2] - a[2] * b_[1],
                a[2] * b_[0] - a[0] * b_[2],
                a[0] * b_[1] - a[1] * b_[0])

    ng = cross(e[0], e[1])
    f0 = (pX[1] - pX[0], pY[1] - pY[0], pZ[1] - pZ[0])
    f1 = (pX[2] - pX[0], pY[2] - pY[0], pZ[2] - pZ[0])
    nd = cross(f0, f1)
    gn = jnp.sqrt(dot(ng, ng))
    dn = jnp.sqrt(dot(nd, nd))
    gn = gn + (gn == 0.0).astype(jnp.float32) * 0.01
    dn = dn + (dn == 0.0).astype(jnp.float32) * 0.01
    lo = (jnp.abs(ng[0] / gn - nd[0] / dn)
          + jnp.abs(ng[1] / gn - nd[1] / dn)
          + jnp.abs(ng[2] / gn - nd[2] / dn))
    return jnp.where(mask, lax.bitcast_convert_type(lo, jnp.int32),
                     jnp.int32(-1))


def _vbits_body(dep, cx, cy, pm, out):
    for b in range(B):
        out[b] = _group_vbits(dep, cx, cy, pm, b)


def _final_body(dep, cx, cy, pm, vb1, out):
    vbits_all = [vb1[b] for b in range(B)]
    for b in range(B):
        vbits_all.append(_group_vbits(dep, cx, cy, pm, b))
    n_valid = jnp.int32(0)
    for vb in vbits_all:
        n_valid = n_valid + jnp.sum((vb >= 0).astype(jnp.int32))
    n_drop = n_valid // 4
    count = n_valid - n_drop

    def step(t, ans):
        cand = ans | lax.shift_left(jnp.int32(1), jnp.int32(30) - t)
        c2 = jnp.int32(0)
        for vb in vbits_all:
            c2 = c2 + jnp.sum((vb >= cand).astype(jnp.int32))
        return jnp.where(c2 >= count, cand, ans)

    ans = lax.fori_loop(0, 31, step, jnp.int32(0))
    cnt_gt = jnp.int32(0)
    sum_gt = jnp.float32(0.0)
    for vb in vbits_all:
        gtm = vb > ans
        lo = lax.bitcast_convert_type(vb, jnp.float32)
        cnt_gt = cnt_gt + jnp.sum(gtm.astype(jnp.int32))
        sum_gt = sum_gt + jnp.sum(jnp.where(gtm, lo, jnp.float32(0.0)))
    tstar = lax.bitcast_convert_type(ans, jnp.float32)
    kept = sum_gt + (count - cnt_gt).astype(jnp.float32) * tstar
    res = kept / count.astype(jnp.float32)
    out[...] = jnp.broadcast_to(res, (1, 1))


def kernel(gt_depth, pred_depth, fx, fy):
    gt2 = gt_depth.reshape(B, HW)
    pr2 = pred_depth.reshape(B, HW)
    dep1, tab = _get_sc_gather(True)(gt2, pr2, jnp.asarray(_PIX1))
    dep2 = _get_sc_gather(False)(tab, jnp.asarray(_PIX2))
    cx1 = jnp.asarray(_CU1) / fx
    cy1 = jnp.asarray(_CV1) / fy
    cx2 = jnp.asarray(_CU2) / fx
    cy2 = jnp.asarray(_CV2) / fy
    vb1 = pl.pallas_call(
        _vbits_body,
        out_shape=jax.ShapeDtypeStruct((B, HGROW, LANES), jnp.int32),
    )(dep1, cx1, cy1, jnp.asarray(_PM1))
    out = pl.pallas_call(
        _final_body,
        out_shape=jax.ShapeDtypeStruct((1, 1), jnp.float32),
    )(dep2, cx2, cy2, jnp.asarray(_PM2), vb1)
    return out[0, 0]


# speedup vs baseline: 1.4752x; 1.4752x over previous
import functools

import numpy as np
import jax
import jax.numpy as jnp
from jax import lax
from jax.experimental import pallas as pl
from jax.experimental.pallas import tpu as pltpu, tpu_sc as plsc

H = 384
W = 384
B = 4
HW = H * W
G = int(HW * 0.15)
LANES = 128
GP = 22528
GROW = GP // LANES
NW = 32
NROWS = 544
ROWS = NROWS // NW
TW = 2 * B
NSUB = 16
PIXSUB = HW // NSUB
PP = PIXSUB // 4

_DELTA_Z = 1e-4
_DELTA_COS = 0.867
_DXYZ = 0.005


def _static_consts():
    rng = np.random.RandomState(0)
    ps = []
    for _ in range(3):
        p = rng.choice(HW, G, replace=True)
        rng.shuffle(p)
        ps.append(p.astype(np.int64))
    pix = np.zeros((NROWS, LANES), np.int32)
    cu = np.zeros((3, GP), np.float32)
    cv = np.zeros((3, GP), np.float32)
    pm = np.zeros((GP,), np.float32)
    pm[:G] = 1.0
    for k in range(3):
        p = ps[k]
        pk = np.zeros((GP,), np.int32)
        pk[:G] = p.astype(np.int32)
        pix[k * GROW:(k + 1) * GROW] = pk.reshape(GROW, LANES)
        cu[k, :G] = (p % W).astype(np.float32) - float(W // 2)
        cv[k, :G] = (p // W).astype(np.float32) - float(H // 2)
    return (pix,
            cu.reshape(3, GROW, LANES),
            cv.reshape(3, GROW, LANES),
            pm.reshape(GROW, LANES))


_PIX, _CU, _CV, _PM = _static_consts()


@functools.lru_cache(maxsize=1)
def _get_sc_gather():
    mesh = plsc.VectorSubcoreMesh(core_axis_name="c", subcore_axis_name="s",
                                  num_cores=2, num_subcores=16)

    @functools.partial(
        pl.kernel,
        out_type=(jax.ShapeDtypeStruct((TW, NROWS, LANES), jnp.float32),
                  jax.ShapeDtypeStruct((2, HW, TW), jnp.float32)),
        mesh=mesh,
        scratch_types=[
            pltpu.VMEM((2, TW, PP), jnp.float32),
            pltpu.VMEM((PP, TW), jnp.float32),
            pltpu.VMEM((ROWS, LANES), jnp.int32),
            pltpu.VMEM((ROWS, LANES, TW), jnp.float32),
            pltpu.VMEM((TW, ROWS, LANES), jnp.float32),
            pltpu.SemaphoreType.DMA,
            pltpu.SemaphoreType.DMA,
            pltpu.SemaphoreType.DMA,
            pltpu.SemaphoreType.DMA,
        ],
        compiler_params=pltpu.CompilerParams(use_tc_tiling_on_sc=False,
                                             needs_layout_passes=False),
    )
    def _sc_gather(gt_hbm, pr_hbm, idx_hbm, out_hbm, tab_hbm,
                   chunks, tblh, idx_v, buf, outv, sem, sem2, sem3, sem4):
        cid = lax.axis_index("c")
        sid = lax.axis_index("s")
        wid = sid * 2 + cid
        rowbase = wid * ROWS
        idx_h = pltpu.async_copy(idx_hbm.at[pl.ds(rowbase, ROWS)], idx_v,
                                 sem2)
        lane = lax.iota(jnp.int32, 16)
        tconst = [jnp.full((16,), t, jnp.int32) for t in range(TW)]

        def fire_loads(p, sel):
            base = sid * PIXSUB + p * PP
            hs = []
            for t in range(TW):
                src = gt_hbm if t < B else pr_hbm
                hs.append(pltpu.async_copy(
                    src.at[t % B, pl.ds(base, PP)], chunks.at[sel, t], sem))
            return hs

        pend = fire_loads(0, 0)
        for p in range(4):
            cur = p % 2
            for hh in pend:
                hh.wait()
            if p < 3:
                pend = fire_loads(p + 1, 1 - cur)

            @plsc.parallel_loop(0, PP // 16, 1, unroll=4)
            def interleave(i):
                pvec = lane + 16 * i
                for t in range(TW):
                    vals = chunks[cur, t, pl.ds(16 * i, 16)]
                    plsc.store_scatter(tblh, [pvec, tconst[t]], vals)
            pltpu.sync_copy(
                tblh, tab_hbm.at[cid, pl.ds(sid * PIXSUB + p * PP, PP)])
        idx_h.wait()
        plsc.subcore_barrier()
        sems = [sem, sem2, sem3, sem4]
        handles = []
        for r in range(ROWS):
            handles.append(
                pltpu.async_copy(tab_hbm.at[cid].at[idx_v.at[r]],
                                 buf.at[r], sems[r % 4]))
        for h in handles:
            h.wait()
        lane = lax.iota(jnp.int32, 16)

        @plsc.parallel_loop(0, ROWS, 1, unroll=2)
        def deint(r):
            rr = jnp.full((16,), r, jnp.int32)
            for t in range(TW):
                tt = jnp.full((16,), t, jnp.int32)
                for j in range(LANES // 16):
                    ll = lane + (16 * j)
                    vals = plsc.load_gather(buf, [rr, ll, tt])
                    outv[t, r, pl.ds(16 * j, 16)] = vals
        pltpu.sync_copy(outv, out_hbm.at[:, pl.ds(rowbase, ROWS)])

    return _sc_gather


def _loss_body(dep, cx, cy, pm, out):
    pmv = pm[...] > 0
    cxs = [cx[k] for k in range(3)]
    cys = [cy[k] for k in range(3)]

    vbits_all = []
    lo_all = []
    n_valid = jnp.int32(0)
    for b in range(B):
        dg = [dep[b, k * GROW:(k + 1) * GROW] for k in range(3)]
        dp = [dep[B + b, k * GROW:(k + 1) * GROW] for k in range(3)]

        gx = [cxs[k] * jnp.abs(dg[k]) for k in range(3)]
        gy = [cys[k] * jnp.abs(dg[k]) for k in range(3)]
        gz = dg

        pX = [cxs[k] * jnp.abs(dp[k]) for k in range(3)]
        pY = [cys[k] * jnp.abs(dp[k]) for k in range(3)]
        pZ = list(dp)
        cz = [dp[j] == 0.0 for j in range(3)]
        pX = [jnp.where(cz[0], jnp.float32(1e-4), pX[k]) for k in range(3)]
        pY = [jnp.where(cz[1], jnp.float32(1e-4), pY[k]) for k in range(3)]
        pZ = [jnp.where(cz[2], jnp.float32(1e-4), pZ[k]) for k in range(3)]

        pairs = ((0, 1), (0, 2), (1, 2))
        e = [(gx[j] - gx[i], gy[j] - gy[i], gz[j] - gz[i]) for (i, j) in pairs]

        def dot(a, b):
            return a[0] * b[0] + a[1] * b[1] + a[2] * b[2]

        d00 = dot(e[0], e[0])
        d11 = dot(e[1], e[1])
        d22 = dot(e[2], e[2])
        d01 = dot(e[0], e[1])
        d02 = dot(e[0], e[2])
        d12 = dot(e[1], e[2])
        n0 = jnp.sqrt(d00)
        n1 = jnp.sqrt(d11)
        n2 = jnp.sqrt(d22)
        eps = jnp.float32(1e-8)

        def hit(num, na, nb):
            return (jnp.abs(num / (na * nb + eps)) > _DELTA_COS).astype(
                jnp.int32)

        cnt = (hit(d00, n0, n0) + hit(d11, n1, n1) + hit(d22, n2, n2)
               + 2 * hit(d01, n0, n1) + 2 * hit(d02, n0, n2)
               + 2 * hit(d12, n1, n2))
        mask_cos = cnt > 3
        mask_pad = ((gz[0] > _DELTA_Z) & (gz[1] > _DELTA_Z)
                    & (gz[2] > _DELTA_Z))
        mx = ((jnp.abs(e[0][0]) < _DXYZ) | (jnp.abs(e[1][0]) < _DXYZ)
              | (jnp.abs(e[2][0]) < _DXYZ))
        my = ((jnp.abs(e[0][1]) < _DXYZ) | (jnp.abs(e[1][1]) < _DXYZ)
              | (jnp.abs(e[2][1]) < _DXYZ))
        mz = ((jnp.abs(e[0][2]) < _DXYZ) | (jnp.abs(e[1][2]) < _DXYZ)
              | (jnp.abs(e[2][2]) < _DXYZ))
        mask = mask_pad & jnp.logical_not((mx & my & mz) | mask_cos) & pmv

        def cross(a, b):
            return (a[1] * b[2] - a[2] * b[1],
                    a[2] * b[0] - a[0] * b[2],
                    a[0] * b[1] - a[1] * b[0])

        ng = cross(e[0], e[1])
        f0 = (pX[1] - pX[0], pY[1] - pY[0], pZ[1] - pZ[0])
        f1 = (pX[2] - pX[0], pY[2] - pY[0], pZ[2] - pZ[0])
        nd = cross(f0, f1)
        gn = jnp.sqrt(dot(ng, ng))
        dn = jnp.sqrt(dot(nd, nd))
        gn = gn + (gn == 0.0).astype(jnp.float32) * 0.01
        dn = dn + (dn == 0.0).astype(jnp.float32) * 0.01
        lo = (jnp.abs(ng[0] / gn - nd[0] / dn)
              + jnp.abs(ng[1] / gn - nd[1] / dn)
              + jnp.abs(ng[2] / gn - nd[2] / dn))

        vbits = jnp.where(mask, lax.bitcast_convert_type(lo, jnp.int32),
                          jnp.int32(-1))
        vbits_all.append(vbits)
        lo_all.append(lo)
        n_valid = n_valid + jnp.sum(mask.astype(jnp.int32))

    n_drop = n_valid // 4
    count = n_valid - n_drop

    def step(t, ans):
        cand = ans | lax.shift_left(jnp.int32(1), jnp.int32(30) - t)
        c2 = jnp.int32(0)
        for vb in vbits_all:
            c2 = c2 + jnp.sum((vb >= cand).astype(jnp.int32))
        return jnp.where(c2 >= count, cand, ans)

    ans = lax.fori_loop(0, 31, step, jnp.int32(0))
    cnt_gt = jnp.int32(0)
    sum_gt = jnp.float32(0.0)
    for vb, lo in zip(vbits_all, lo_all):
        gtm = vb > ans
        cnt_gt = cnt_gt + jnp.sum(gtm.astype(jnp.int32))
        sum_gt = sum_gt + jnp.sum(jnp.where(gtm, lo, jnp.float32(0.0)))
    tstar = lax.bitcast_convert_type(ans, jnp.float32)
    kept = sum_gt + (count - cnt_gt).astype(jnp.float32) * tstar
    res = kept / count.astype(jnp.float32)
    out[...] = jnp.broadcast_to(res, (1, 1))


def kernel(gt_depth, pred_depth, fx, fy):
    dep_rows, _ = _get_sc_gather()(gt_depth.reshape(B, HW),
                                   pred_depth.reshape(B, HW),
                                   jnp.asarray(_PIX))
    dep = dep_rows
    cx = jnp.asarray(_CU) / fx
    cy = jnp.asarray(_CV) / fy
    out = pl.pallas_call(
        _loss_body,
        out_shape=jax.ShapeDtypeStruct((1, 1), jnp.float32),
    )(dep, cx, cy, jnp.asarray(_PM))
    return out[0, 0]
